# Initial kernel scaffold; baseline (speedup 1.0000x reference)
import jax
import jax.numpy as jnp
from jax.experimental import pallas as pl


def kernel(x_bits, luts1, luts2, W, mapping1, mapping2):
    raise NotImplementedError("write your pallas kernel here")


# speedup vs baseline: 215.1105x; 215.1105x over previous
import functools

import jax
import jax.numpy as jnp
from jax import lax
from jax.experimental import pallas as pl
from jax.experimental.pallas import tpu as pltpu
from jax.experimental.pallas import tpu_sc as plsc

B = 4096
IN_BITS = 3072
N1 = 2000
N2 = 1000
K = 6
C = 1000
N1P = 2048
N2P = 1024

NC = 2
NS = 16
NW = NC * NS
L = 16

BB = 512


def _build_mt(map_ref, m_scr, n_rows, n_in):
    rc = 256
    iota = lax.broadcasted_iota(jnp.int32, (rc, n_in), 1)

    def chunk(i, carry):
        acc = jnp.zeros((rc, n_in), jnp.float32)
        for j in range(K):
            col = map_ref[pl.ds(i * rc, rc), j : j + 1]
            acc += jnp.where(iota == col, float(2**j), 0.0)
        m_scr[pl.ds(i * rc, rc), :] = acc.astype(jnp.bfloat16)
        return carry

    lax.fori_loop(0, n_rows // rc, chunk, 0)


def _mm1_body(map_ref, x_ref, out_ref, m_scr):
    @pl.when(pl.program_id(0) == 0)
    def _():
        _build_mt(map_ref, m_scr, N1P, IN_BITS)

    x = x_ref[...]
    for n0 in range(0, N1P, 256):
        acc = lax.dot_general(
            m_scr[n0 : n0 + 256, :], x, (((1,), (0,)), ((), ())),
            preferred_element_type=jnp.float32)
        out_ref[n0 : n0 + 256, :] = acc.astype(jnp.int32)


def _mm1(m1p, x_t):
    return pl.pallas_call(
        _mm1_body,
        grid=(B // BB,),
        in_specs=[
            pl.BlockSpec((N1P, K), lambda b: (0, 0)),
            pl.BlockSpec((IN_BITS, BB), lambda b: (0, b)),
        ],
        out_specs=pl.BlockSpec((N1P, BB), lambda b: (0, b)),
        out_shape=jax.ShapeDtypeStruct((N1P, B), jnp.int32),
        scratch_shapes=[pltpu.VMEM((N1P, IN_BITS), jnp.bfloat16)],
    )(m1p, x_t)


def _mm2_body(map_ref, bits_ref, out_ref, m_scr):
    @pl.when(pl.program_id(0) == 0)
    def _():
        _build_mt(map_ref, m_scr, N2P, N1P)

    bb = bits_ref[...].astype(jnp.bfloat16)
    for n0 in range(0, N2P, 512):
        acc = lax.dot_general(
            m_scr[n0 : n0 + 512, :], bb, (((1,), (0,)), ((), ())),
            preferred_element_type=jnp.float32)
        out_ref[n0 : n0 + 512, :] = acc.astype(jnp.int32)


def _mm2(m2p, bits1):
    return pl.pallas_call(
        _mm2_body,
        grid=(B // BB,),
        in_specs=[
            pl.BlockSpec((N2P, K), lambda b: (0, 0)),
            pl.BlockSpec((N1P, BB), lambda b: (0, b)),
        ],
        out_specs=pl.BlockSpec((N2P, BB), lambda b: (0, b)),
        out_shape=jax.ShapeDtypeStruct((N2P, B), jnp.int32),
        scratch_shapes=[pltpu.VMEM((N2P, N1P), jnp.bfloat16)],
    )(m2p, bits1)


def _mm3_body(w_ref, h_ref, out_ref):
    out_ref[...] = lax.dot_general(
        w_ref[...], h_ref[...], (((1,), (0,)), ((), ())),
        preferred_element_type=jnp.float32)


def _mm3(h2, w_pad):
    return pl.pallas_call(
        _mm3_body,
        grid=(B // BB,),
        in_specs=[
            pl.BlockSpec((C, N2P), lambda b: (0, 0)),
            pl.BlockSpec((N2P, BB), lambda b: (0, b)),
        ],
        out_specs=pl.BlockSpec((C, BB), lambda b: (0, b)),
        out_shape=jax.ShapeDtypeStruct((C, B), jnp.float32),
    )(w_pad, h2)


CHUNK = 32768
GROUPS = CHUNK // L
GPR = B // L
UNROLL = 8


def _make_sc_lut(n_rows, sigmoid):
    rpw = n_rows // NW
    span = rpw * B
    n_chunks = span // CHUNK
    mesh = plsc.VectorSubcoreMesh(core_axis_name="c", subcore_axis_name="s")

    @functools.partial(
        pl.kernel,
        out_type=jax.ShapeDtypeStruct((n_rows * B,), jnp.float32),
        mesh=mesh,
        scratch_types=[
            pltpu.VMEM((rpw * 64,), jnp.float32),
            pltpu.VMEM((CHUNK,), jnp.int32),
            pltpu.VMEM((CHUNK,), jnp.float32),
        ],
        compiler_params=pltpu.CompilerParams(needs_layout_passes=False),
    )
    def k(addr_hbm, luts_hbm, out_hbm, luts_v, addr_v, out_v):
        wid = lax.axis_index("s") * NC + lax.axis_index("c")
        row0 = wid * rpw
        pltpu.sync_copy(luts_hbm.at[pl.ds(row0 * 64, rpw * 64)], luts_v)
        base = wid * span
        for c in range(n_chunks):
            pltpu.sync_copy(addr_hbm.at[pl.ds(base + c * CHUNK, CHUNK)], addr_v)

            def body(g, carry, c=c):
                for u in range(UNROLL):
                    gg = g * UNROLL + u
                    off = gg * L
                    lrow = (c * GROUPS + gg) >> 8
                    a = addr_v[pl.ds(off, L)]
                    idx = a + (lrow << 6)
                    v = plsc.load_gather(luts_v, [idx])
                    if sigmoid:
                        o = 1.0 / (1.0 + jnp.exp(-v))
                    else:
                        o = jnp.where(v >= 0.0, 1.0, 0.0).astype(jnp.float32)
                    out_v[pl.ds(off, L)] = o
                return carry

            lax.fori_loop(0, GROUPS // UNROLL, body, 0)
            pltpu.sync_copy(out_v, out_hbm.at[pl.ds(base + c * CHUNK, CHUNK)])

    return k


_sc_lut1 = _make_sc_lut(N1P, sigmoid=False)
_sc_lut2 = _make_sc_lut(N2P, sigmoid=True)


def kernel(x_bits, luts1, luts2, W, mapping1, mapping2):
    x_t = x_bits.astype(jnp.bfloat16).T
    m1p = jnp.pad(mapping1, ((0, N1P - N1), (0, 0)))
    m2p = jnp.pad(mapping2, ((0, N2P - N2), (0, 0)))
    l1p = jnp.pad(luts1, ((0, N1P - N1), (0, 0))).reshape(-1)
    l2p = jnp.pad(luts2, ((0, N2P - N2), (0, 0))).reshape(-1)
    w_pad = jnp.pad(W, ((0, 0), (0, N2P - N2)))

    a1 = _mm1(m1p, x_t)
    bits1 = _sc_lut1(a1.reshape(-1), l1p).reshape(N1P, B)
    a2 = _mm2(m2p, bits1)
    h2 = _sc_lut2(a2.reshape(-1), l2p).reshape(N2P, B)
    return _mm3(h2, w_pad).T


# speedup vs baseline: 423.5591x; 1.9690x over previous
import functools

import jax
import jax.numpy as jnp
from jax import lax
from jax.experimental import pallas as pl
from jax.experimental.pallas import tpu as pltpu
from jax.experimental.pallas import tpu_sc as plsc

B = 4096
IN_BITS = 3072
N1 = 2000
N2 = 1000
K = 6
C = 1000
N1P = 2048
N2P = 1024

NC = 2
NS = 16
NW = NC * NS
L = 16

BB = 512


def _build_mt(map_ref, m_scr, n_rows, n_in):
    rc = 256
    iota = lax.broadcasted_iota(jnp.int32, (rc, n_in), 1)

    def chunk(i, carry):
        acc = jnp.zeros((rc, n_in), jnp.float32)
        for j in range(K):
            col = map_ref[pl.ds(i * rc, rc), j : j + 1]
            acc += jnp.where(iota == col, float(2**j), 0.0)
        m_scr[pl.ds(i * rc, rc), :] = acc.astype(jnp.bfloat16)
        return carry

    lax.fori_loop(0, n_rows // rc, chunk, 0)


def _mm1_body(map_ref, x_ref, out_ref, m_scr):
    @pl.when(pl.program_id(0) == 0)
    def _():
        _build_mt(map_ref, m_scr, N1P, IN_BITS)

    x = x_ref[...]
    for n0 in range(0, N1P, 256):
        acc = lax.dot_general(
            m_scr[n0 : n0 + 256, :], x, (((1,), (0,)), ((), ())),
            preferred_element_type=jnp.float32)
        out_ref[n0 : n0 + 256, :] = acc.astype(jnp.int32)


def _mm1(m1p, x_t):
    return pl.pallas_call(
        _mm1_body,
        grid=(B // BB,),
        in_specs=[
            pl.BlockSpec((N1P, K), lambda b: (0, 0)),
            pl.BlockSpec((IN_BITS, BB), lambda b: (0, b)),
        ],
        out_specs=pl.BlockSpec((N1P, BB), lambda b: (0, b)),
        out_shape=jax.ShapeDtypeStruct((N1P, B), jnp.int32),
        scratch_shapes=[pltpu.VMEM((N1P, IN_BITS), jnp.bfloat16)],
    )(m1p, x_t)


def _mm2_body(map_ref, bits_ref, out_ref, m_scr):
    @pl.when(pl.program_id(0) == 0)
    def _():
        _build_mt(map_ref, m_scr, N2P, N1P)

    bb = bits_ref[...].astype(jnp.bfloat16)
    for n0 in range(0, N2P, 512):
        acc = lax.dot_general(
            m_scr[n0 : n0 + 512, :], bb, (((1,), (0,)), ((), ())),
            preferred_element_type=jnp.float32)
        out_ref[n0 : n0 + 512, :] = acc.astype(jnp.int32)


def _mm2(m2p, bits1):
    return pl.pallas_call(
        _mm2_body,
        grid=(B // BB,),
        in_specs=[
            pl.BlockSpec((N2P, K), lambda b: (0, 0)),
            pl.BlockSpec((N1P, BB), lambda b: (0, b)),
        ],
        out_specs=pl.BlockSpec((N2P, BB), lambda b: (0, b)),
        out_shape=jax.ShapeDtypeStruct((N2P, B), jnp.int32),
        scratch_shapes=[pltpu.VMEM((N2P, N1P), jnp.bfloat16)],
    )(m2p, bits1)


def _mm3_body(w_ref, h_ref, out_ref):
    out_ref[...] = lax.dot_general(
        w_ref[...], h_ref[...], (((1,), (0,)), ((), ())),
        preferred_element_type=jnp.float32)


def _mm3(h2, w_pad):
    return pl.pallas_call(
        _mm3_body,
        grid=(B // BB,),
        in_specs=[
            pl.BlockSpec((C, N2P), lambda b: (0, 0)),
            pl.BlockSpec((N2P, BB), lambda b: (0, b)),
        ],
        out_specs=pl.BlockSpec((C, BB), lambda b: (0, b)),
        out_shape=jax.ShapeDtypeStruct((C, B), jnp.float32),
    )(w_pad, h2)


CHUNK = 32768
GROUPS = CHUNK // L
GPR = B // L
UNROLL = 8


def _make_sc_lut(n_rows, sigmoid):
    rpw = n_rows // NW
    span = rpw * B
    n_chunks = span // CHUNK
    mesh = plsc.VectorSubcoreMesh(core_axis_name="c", subcore_axis_name="s")

    @functools.partial(
        pl.kernel,
        out_type=jax.ShapeDtypeStruct((n_rows * B,), jnp.float32),
        mesh=mesh,
        scratch_types=[
            pltpu.VMEM((rpw * 64,), jnp.float32),
            pltpu.VMEM((CHUNK,), jnp.int32),
            pltpu.VMEM((CHUNK,), jnp.float32),
        ],
        compiler_params=pltpu.CompilerParams(needs_layout_passes=False),
    )
    def k(addr_hbm, luts_hbm, out_hbm, luts_v, addr_v, out_v):
        wid = lax.axis_index("s") * NC + lax.axis_index("c")
        row0 = wid * rpw
        pltpu.sync_copy(luts_hbm.at[pl.ds(row0 * 64, rpw * 64)], luts_v)
        base = wid * span
        for c in range(n_chunks):
            pltpu.sync_copy(addr_hbm.at[pl.ds(base + c * CHUNK, CHUNK)], addr_v)

            @plsc.parallel_loop(0, GROUPS, 1, unroll=UNROLL)
            def body(gg, c=c):
                off = gg * L
                lrow = (c * GROUPS + gg) >> 8
                a = addr_v[pl.ds(off, L)]
                idx = a + (lrow << 6)
                v = plsc.load_gather(luts_v, [idx])
                if sigmoid:
                    o = 1.0 / (1.0 + jnp.exp(-v))
                else:
                    o = jnp.where(v >= 0.0, 1.0, 0.0).astype(jnp.float32)
                out_v[pl.ds(off, L)] = o

            pltpu.sync_copy(out_v, out_hbm.at[pl.ds(base + c * CHUNK, CHUNK)])

    return k


_sc_lut1 = _make_sc_lut(N1P, sigmoid=False)
_sc_lut2 = _make_sc_lut(N2P, sigmoid=True)


def kernel(x_bits, luts1, luts2, W, mapping1, mapping2):
    x_t = x_bits.astype(jnp.bfloat16).T
    m1p = jnp.pad(mapping1, ((0, N1P - N1), (0, 0)))
    m2p = jnp.pad(mapping2, ((0, N2P - N2), (0, 0)))
    l1p = jnp.pad(luts1, ((0, N1P - N1), (0, 0))).reshape(-1)
    l2p = jnp.pad(luts2, ((0, N2P - N2), (0, 0))).reshape(-1)
    w_pad = jnp.pad(W, ((0, 0), (0, N2P - N2)))

    a1 = _mm1(m1p, x_t)
    bits1 = _sc_lut1(a1.reshape(-1), l1p).reshape(N1P, B)
    a2 = _mm2(m2p, bits1)
    h2 = _sc_lut2(a2.reshape(-1), l2p).reshape(N2P, B)
    return _mm3(h2, w_pad).T


# speedup vs baseline: 583.5451x; 1.3777x over previous
import functools

import jax
import jax.numpy as jnp
from jax import lax
from jax.experimental import pallas as pl
from jax.experimental.pallas import tpu as pltpu
from jax.experimental.pallas import tpu_sc as plsc

B = 4096
IN_BITS = 3072
N1 = 2000
N2 = 1000
K = 6
C = 1000
N1P = 2048
N2P = 1024

NC = 2
NS = 16
NW = NC * NS
L = 16

BB = 512


def _build_mt(map_ref, m_scr, n_rows, n_in):
    rc = 256
    iota = lax.broadcasted_iota(jnp.int32, (rc, n_in), 1)

    def chunk(i, carry):
        acc = jnp.zeros((rc, n_in), jnp.float32)
        for j in range(K):
            col = map_ref[pl.ds(i * rc, rc), j : j + 1]
            acc += jnp.where(iota == col, float(2**j), 0.0)
        m_scr[pl.ds(i * rc, rc), :] = acc.astype(jnp.bfloat16)
        return carry

    lax.fori_loop(0, n_rows // rc, chunk, 0)


def _mm1_body(map_ref, x_ref, out_ref, m_scr):
    @pl.when(pl.program_id(0) == 0)
    def _():
        _build_mt(map_ref, m_scr, N1P, IN_BITS)

    x = x_ref[...]
    for n0 in range(0, N1P, 256):
        acc = lax.dot_general(
            m_scr[n0 : n0 + 256, :], x, (((1,), (0,)), ((), ())),
            preferred_element_type=jnp.float32)
        out_ref[n0 : n0 + 256, :] = acc.astype(jnp.int32)


def _mm1(m1p, x_t):
    return pl.pallas_call(
        _mm1_body,
        grid=(B // BB,),
        in_specs=[
            pl.BlockSpec((N1P, K), lambda b: (0, 0)),
            pl.BlockSpec((IN_BITS, BB), lambda b: (0, b)),
        ],
        out_specs=pl.BlockSpec((N1P, BB), lambda b: (0, b)),
        out_shape=jax.ShapeDtypeStruct((N1P, B), jnp.int32),
        scratch_shapes=[pltpu.VMEM((N1P, IN_BITS), jnp.bfloat16)],
    )(m1p, x_t)


def _mm2_body(map_ref, bits_ref, out_ref, m_scr):
    @pl.when(pl.program_id(0) == 0)
    def _():
        _build_mt(map_ref, m_scr, N2P, N1P)

    bb = bits_ref[...].astype(jnp.bfloat16)
    for n0 in range(0, N2P, 512):
        acc = lax.dot_general(
            m_scr[n0 : n0 + 512, :], bb, (((1,), (0,)), ((), ())),
            preferred_element_type=jnp.float32)
        out_ref[n0 : n0 + 512, :] = acc.astype(jnp.int32)


def _mm2(m2p, bits1):
    return pl.pallas_call(
        _mm2_body,
        grid=(B // BB,),
        in_specs=[
            pl.BlockSpec((N2P, K), lambda b: (0, 0)),
            pl.BlockSpec((N1P, BB), lambda b: (0, b)),
        ],
        out_specs=pl.BlockSpec((N2P, BB), lambda b: (0, b)),
        out_shape=jax.ShapeDtypeStruct((N2P, B), jnp.int32),
        scratch_shapes=[pltpu.VMEM((N2P, N1P), jnp.bfloat16)],
    )(m2p, bits1)


def _mm3_body(w_ref, h_ref, out_ref):
    out_ref[...] = lax.dot_general(
        w_ref[...], h_ref[...].astype(jnp.bfloat16), (((1,), (0,)), ((), ())),
        preferred_element_type=jnp.float32)


def _mm3(h2, w_pad):
    return pl.pallas_call(
        _mm3_body,
        grid=(B // BB,),
        in_specs=[
            pl.BlockSpec((C, N2P), lambda b: (0, 0)),
            pl.BlockSpec((N2P, BB), lambda b: (0, b)),
        ],
        out_specs=pl.BlockSpec((C, BB), lambda b: (0, b)),
        out_shape=jax.ShapeDtypeStruct((C, B), jnp.float32),
    )(w_pad, h2)


CHUNK = 32768
GROUPS = CHUNK // L
GPR = B // L
UNROLL = 8


def _make_sc_lut(n_rows, sigmoid):
    rpw = n_rows // NW
    rg = CHUNK // B
    n_chunks = rpw // rg
    mesh = plsc.VectorSubcoreMesh(core_axis_name="c", subcore_axis_name="s")

    @functools.partial(
        pl.kernel,
        out_type=jax.ShapeDtypeStruct((n_rows, B), jnp.float32),
        mesh=mesh,
        scratch_types=[
            pltpu.VMEM((rpw * 64,), jnp.float32),
            pltpu.VMEM((rg, B), jnp.int32),
            pltpu.VMEM((rg, B), jnp.float32),
        ],
        compiler_params=pltpu.CompilerParams(needs_layout_passes=False),
    )
    def k(addr_hbm, luts_hbm, out_hbm, luts_v, addr_v, out_v):
        wid = lax.axis_index("s") * NC + lax.axis_index("c")
        row0 = wid * rpw
        pltpu.sync_copy(luts_hbm.at[pl.ds(row0 * 64, rpw * 64)], luts_v)
        for c in range(n_chunks):
            pltpu.sync_copy(addr_hbm.at[pl.ds(row0 + c * rg, rg)], addr_v)

            @plsc.parallel_loop(0, GROUPS, 1, unroll=UNROLL)
            def body(gg, c=c):
                r = gg >> 8
                col = (gg & (GPR - 1)) * L
                lrow = c * rg + r
                a = addr_v[r, pl.ds(col, L)]
                idx = a + (lrow << 6)
                v = plsc.load_gather(luts_v, [idx])
                if sigmoid:
                    o = 1.0 / (1.0 + jnp.exp(-v))
                else:
                    o = jnp.where(v >= 0.0, 1.0, 0.0).astype(jnp.float32)
                out_v[r, pl.ds(col, L)] = o

            pltpu.sync_copy(out_v, out_hbm.at[pl.ds(row0 + c * rg, rg)])

    return k


_sc_lut1 = _make_sc_lut(N1P, sigmoid=False)
_sc_lut2 = _make_sc_lut(N2P, sigmoid=True)


def kernel(x_bits, luts1, luts2, W, mapping1, mapping2):
    x_t = x_bits.astype(jnp.bfloat16).T
    m1p = jnp.pad(mapping1, ((0, N1P - N1), (0, 0)))
    m2p = jnp.pad(mapping2, ((0, N2P - N2), (0, 0)))
    l1p = jnp.pad(luts1, ((0, N1P - N1), (0, 0))).reshape(-1)
    l2p = jnp.pad(luts2, ((0, N2P - N2), (0, 0))).reshape(-1)
    w_pad = jnp.pad(W, ((0, 0), (0, N2P - N2))).astype(jnp.bfloat16)

    a1 = _mm1(m1p, x_t)
    bits1 = _sc_lut1(a1, l1p)
    a2 = _mm2(m2p, bits1)
    h2 = _sc_lut2(a2, l2p)
    return _mm3(h2, w_pad).T


# speedup vs baseline: 613.5390x; 1.0514x over previous
import functools

import jax
import jax.numpy as jnp
from jax import lax
from jax.experimental import pallas as pl
from jax.experimental.pallas import tpu as pltpu
from jax.experimental.pallas import tpu_sc as plsc

B = 4096
IN_BITS = 3072
N1 = 2000
N2 = 1000
K = 6
C = 1000
N1P = 2048
N2P = 1024

NH = 2
BH = B // NH

NC = 2
NS = 16
NW = NC * NS
L = 16

BB = 512


def _build_mt(map_ref, m_scr, n_rows, n_in):
    rc = 256
    iota = lax.broadcasted_iota(jnp.int32, (rc, n_in), 1)

    def chunk(i, carry):
        acc = jnp.zeros((rc, n_in), jnp.float32)
        for j in range(K):
            col = map_ref[pl.ds(i * rc, rc), j : j + 1]
            acc += jnp.where(iota == col, float(2**j), 0.0)
        m_scr[pl.ds(i * rc, rc), :] = acc.astype(jnp.bfloat16)
        return carry

    lax.fori_loop(0, n_rows // rc, chunk, 0)


def _mm1_body(map_ref, x_ref, out_ref, m_scr):
    @pl.when(pl.program_id(0) == 0)
    def _():
        _build_mt(map_ref, m_scr, N1P, IN_BITS)

    x = x_ref[...]
    for n0 in range(0, N1P, 256):
        acc = lax.dot_general(
            m_scr[n0 : n0 + 256, :], x, (((1,), (0,)), ((), ())),
            preferred_element_type=jnp.float32)
        out_ref[n0 : n0 + 256, :] = acc.astype(jnp.int32)


def _mm1(m1p, x_t, h):
    return pl.pallas_call(
        _mm1_body,
        grid=(BH // BB,),
        in_specs=[
            pl.BlockSpec((N1P, K), lambda b: (0, 0)),
            pl.BlockSpec((IN_BITS, BB), lambda b, h=h: (0, b + h * (BH // BB))),
        ],
        out_specs=pl.BlockSpec((N1P, BB), lambda b: (0, b)),
        out_shape=jax.ShapeDtypeStruct((N1P, BH), jnp.int32),
        scratch_shapes=[pltpu.VMEM((N1P, IN_BITS), jnp.bfloat16)],
    )(m1p, x_t)


def _mm2_body(map_ref, bits_ref, out_ref, m_scr):
    @pl.when(pl.program_id(0) == 0)
    def _():
        _build_mt(map_ref, m_scr, N2P, N1P)

    bb = bits_ref[...].astype(jnp.bfloat16)
    for n0 in range(0, N2P, 512):
        acc = lax.dot_general(
            m_scr[n0 : n0 + 512, :], bb, (((1,), (0,)), ((), ())),
            preferred_element_type=jnp.float32)
        out_ref[n0 : n0 + 512, :] = acc.astype(jnp.int32)


def _mm2(m2p, bits1):
    return pl.pallas_call(
        _mm2_body,
        grid=(BH // BB,),
        in_specs=[
            pl.BlockSpec((N2P, K), lambda b: (0, 0)),
            pl.BlockSpec((N1P, BB), lambda b: (0, b)),
        ],
        out_specs=pl.BlockSpec((N2P, BB), lambda b: (0, b)),
        out_shape=jax.ShapeDtypeStruct((N2P, BH), jnp.int32),
        scratch_shapes=[pltpu.VMEM((N2P, N1P), jnp.bfloat16)],
    )(m2p, bits1)


def _mm3_body(w_ref, h_ref, out_ref):
    out_ref[...] = lax.dot_general(
        w_ref[...], h_ref[...].astype(jnp.bfloat16), (((1,), (0,)), ((), ())),
        preferred_element_type=jnp.float32)


def _mm3(h2, w_pad):
    return pl.pallas_call(
        _mm3_body,
        grid=(BH // BB,),
        in_specs=[
            pl.BlockSpec((C, N2P), lambda b: (0, 0)),
            pl.BlockSpec((N2P, BB), lambda b: (0, b)),
        ],
        out_specs=pl.BlockSpec((C, BB), lambda b: (0, b)),
        out_shape=jax.ShapeDtypeStruct((C, BH), jnp.float32),
    )(w_pad, h2)


CHUNK = 32768
GROUPS = CHUNK // L
GPR_SHIFT = (BH // L).bit_length() - 1
UNROLL = 8


def _make_sc_lut(n_rows, sigmoid):
    rpw = n_rows // NW
    rg = CHUNK // BH
    n_chunks = rpw // rg
    mesh = plsc.VectorSubcoreMesh(core_axis_name="c", subcore_axis_name="s")

    @functools.partial(
        pl.kernel,
        out_type=jax.ShapeDtypeStruct((n_rows, BH), jnp.float32),
        mesh=mesh,
        scratch_types=[
            pltpu.VMEM((rpw * 64,), jnp.float32),
            pltpu.VMEM((rg, BH), jnp.int32),
            pltpu.VMEM((rg, BH), jnp.float32),
        ],
        compiler_params=pltpu.CompilerParams(needs_layout_passes=False),
    )
    def k(addr_hbm, luts_hbm, out_hbm, luts_v, addr_v, out_v):
        wid = lax.axis_index("s") * NC + lax.axis_index("c")
        row0 = wid * rpw
        pltpu.sync_copy(luts_hbm.at[pl.ds(row0 * 64, rpw * 64)], luts_v)
        for c in range(n_chunks):
            pltpu.sync_copy(addr_hbm.at[pl.ds(row0 + c * rg, rg)], addr_v)

            @plsc.parallel_loop(0, GROUPS, 1, unroll=UNROLL)
            def body(gg, c=c):
                r = gg >> GPR_SHIFT
                col = (gg & ((1 << GPR_SHIFT) - 1)) * L
                lrow = c * rg + r
                a = addr_v[r, pl.ds(col, L)]
                idx = a + (lrow << 6)
                v = plsc.load_gather(luts_v, [idx])
                if sigmoid:
                    o = 1.0 / (1.0 + jnp.exp(-v))
                else:
                    o = jnp.where(v >= 0.0, 1.0, 0.0).astype(jnp.float32)
                out_v[r, pl.ds(col, L)] = o

            pltpu.sync_copy(out_v, out_hbm.at[pl.ds(row0 + c * rg, rg)])

    return k


_sc_lut1 = _make_sc_lut(N1P, sigmoid=False)
_sc_lut2 = _make_sc_lut(N2P, sigmoid=True)


def kernel(x_bits, luts1, luts2, W, mapping1, mapping2):
    x_t = x_bits.astype(jnp.bfloat16).T
    m1p = jnp.pad(mapping1, ((0, N1P - N1), (0, 0)))
    m2p = jnp.pad(mapping2, ((0, N2P - N2), (0, 0)))
    l1p = jnp.pad(luts1, ((0, N1P - N1), (0, 0))).reshape(-1)
    l2p = jnp.pad(luts2, ((0, N2P - N2), (0, 0))).reshape(-1)
    w_pad = jnp.pad(W, ((0, 0), (0, N2P - N2))).astype(jnp.bfloat16)

    halves = []
    for h in range(NH):
        a1 = _mm1(m1p, x_t, h)
        bits1 = _sc_lut1(a1, l1p)
        a2 = _mm2(m2p, bits1)
        h2 = _sc_lut2(a2, l2p)
        halves.append(_mm3(h2, w_pad))
    return jnp.concatenate(halves, axis=1).T
